# baseline (speedup 1.0000x reference)
import functools

import jax
import jax.numpy as jnp
from jax import lax
from jax.experimental import pallas as pl
from jax.experimental.pallas import tpu as pltpu
from jax.experimental.pallas import tpu_sc as plsc

NC = 2
NS = 16
NW = NC * NS

B1 = 16384
SL = 50
D = 64

CB = 8
RP = CB * SL
BW = B1 // NW
NCH = BW // CB


@jax.jit
def _sc_lookup(table2, idx1d):
    mesh = plsc.VectorSubcoreMesh(
        core_axis_name="c", subcore_axis_name="s",
        num_cores=NC, num_subcores=NS)

    @functools.partial(
        pl.kernel,
        out_type=jax.ShapeDtypeStruct((B1, SL, D), jnp.float32),
        mesh=mesh,
        scratch_types=[
            pltpu.VMEM((RP,), jnp.int32),
            pltpu.VMEM((RP,), jnp.int32),
            pltpu.VMEM((RP,), jnp.int32),
            pltpu.VMEM((RP, 2 * D), jnp.float32),
            pltpu.VMEM((RP, D), jnp.float32),
            pltpu.SemaphoreType.DMA,
        ],
    )
    def k(table_hbm, idx_hbm, out_hbm, idx_v, widx_v, poff_v,
          g_v, out_v, sem):
        wid = lax.axis_index("s") * NC + lax.axis_index("c")
        rbase = wid * BW * SL
        bbase = wid * BW

        def chunk(ci, carry):
            roff = rbase + ci * RP
            pltpu.sync_copy(idx_hbm.at[pl.ds(roff, RP)], idx_v)

            def vcomp(vi, c):
                v = idx_v[pl.ds(vi * 16, 16)]
                widx_v[pl.ds(vi * 16, 16)] = lax.shift_right_logical(v, 1)
                poff_v[pl.ds(vi * 16, 16)] = (v & 1) * D
                return c

            lax.fori_loop(0, RP // 16, vcomp, 0)
            pltpu.async_copy(table_hbm.at[widx_v], g_v, sem).wait()

            def sel16(g, c):
                poff16 = poff_v[pl.ds(g * 16, 16)]
                for u in range(16):
                    r = g * 16 + u
                    off = poff16[u]
                    for q in range(D // 16):
                        out_v[r, pl.ds(q * 16, 16)] = (
                            g_v[r, pl.ds(off + q * 16, 16)])
                return c

            lax.fori_loop(0, RP // 16, sel16, 0)
            for b in range(CB):
                pltpu.sync_copy(out_v.at[pl.ds(b * SL, SL)],
                                out_hbm.at[bbase + ci * CB + b])
            return carry

        lax.fori_loop(0, NCH, chunk, 0)

    return k(table2, idx1d)


def kernel(embedds, input):
    table2 = embedds.reshape(embedds.shape[0] // 2, 2 * D)
    idx1d = input.reshape(-1).astype(jnp.int32)
    return _sc_lookup(table2, idx1d)
